# baseline (speedup 1.0000x reference)
import functools

import jax
import jax.numpy as jnp
from jax import lax
from jax.experimental import pallas as pl
from jax.experimental.pallas import tpu as pltpu
from jax.experimental.pallas import tpu_sc as plsc

N_TYPES = 17
D_HALF = 512
TM = 256
NB = 213
MP = NB * TM

_NC = 2
_NS = 16
_NW = _NC * _NS

_CHD = 56
_CPWD = 28
_CHC = 112
_CPWC = 14

_RANK_S = 128


def _dispatch(x, edge_attr, slot):
    n = x.shape[0]
    mesh = plsc.VectorSubcoreMesh(core_axis_name="c", subcore_axis_name="s")

    @functools.partial(
        pl.kernel,
        mesh=mesh,
        out_type=(
            jax.ShapeDtypeStruct((MP, D_HALF), jnp.float32),
            jax.ShapeDtypeStruct((MP, D_HALF), jnp.float32),
        ),
        scratch_types=[
            pltpu.VMEM((_CHD,), jnp.int32),
            pltpu.VMEM((_CHD,), jnp.int32),
            pltpu.VMEM((_CHD, D_HALF), jnp.float32),
            pltpu.VMEM((_CHD, D_HALF), jnp.float32),
            pltpu.VMEM((_CHD, D_HALF), jnp.float32),
            pltpu.VMEM((_CHD, D_HALF), jnp.float32),
            pltpu.SemaphoreType.DMA,
            pltpu.SemaphoreType.DMA,
            pltpu.SemaphoreType.DMA,
            pltpu.SemaphoreType.DMA,
        ],
    )
    def dispatch_kernel(x_hbm, e_hbm, idx_hbm, sx_hbm, se_hbm,
                        iv0, iv1, rx0, rx1, re0, re1, sx0, sx1, se0, se1):
        wid = lax.axis_index("s") * _NC + lax.axis_index("c")
        iv, rx, re = [iv0, iv1], [rx0, rx1], [re0, re1]
        ssx, sse = [sx0, sx1], [se0, se1]

        def step(k, _):
            for i in (0, 1):
                q = wid * _CPWD + 2 * k + i
                start = jnp.minimum(q * _CHD, n - _CHD)

                @pl.when(k > 0)
                def _():
                    pltpu.make_async_copy(rx[i], sx_hbm.at[iv[i]], ssx[i]).wait()
                    pltpu.make_async_copy(re[i], se_hbm.at[iv[i]], sse[i]).wait()

                pltpu.sync_copy(idx_hbm.at[pl.ds(start, _CHD)], iv[i])
                pltpu.sync_copy(x_hbm.at[pl.ds(start, _CHD)], rx[i])
                pltpu.sync_copy(e_hbm.at[pl.ds(start, _CHD)], re[i])
                pltpu.async_copy(rx[i], sx_hbm.at[iv[i]], ssx[i])
                pltpu.async_copy(re[i], se_hbm.at[iv[i]], sse[i])
            return ()

        lax.fori_loop(0, _CPWD // 2, step, ())
        for i in (0, 1):
            pltpu.make_async_copy(rx[i], sx_hbm.at[iv[i]], ssx[i]).wait()
            pltpu.make_async_copy(re[i], se_hbm.at[iv[i]], sse[i]).wait()

    return dispatch_kernel(x, edge_attr, slot)


def _collect(table, slot, n):
    mesh = plsc.VectorSubcoreMesh(core_axis_name="c", subcore_axis_name="s")

    @functools.partial(
        pl.kernel,
        mesh=mesh,
        out_type=jax.ShapeDtypeStruct((n, D_HALF), jnp.float32),
        scratch_types=[
            pltpu.VMEM((_CHC,), jnp.int32),
            pltpu.VMEM((_CHC,), jnp.int32),
            pltpu.VMEM((_CHC, D_HALF), jnp.float32),
            pltpu.VMEM((_CHC, D_HALF), jnp.float32),
            pltpu.SemaphoreType.DMA,
            pltpu.SemaphoreType.DMA,
            pltpu.SemaphoreType.DMA,
            pltpu.SemaphoreType.DMA,
        ],
    )
    def collect_kernel(table_hbm, idx_hbm, out_hbm,
                       iv0, iv1, rv0, rv1, g0, g1, w0, w1):
        wid = lax.axis_index("s") * _NC + lax.axis_index("c")
        iv, rv = [iv0, iv1], [rv0, rv1]
        gs, ws = [g0, g1], [w0, w1]

        def chunk_start(k, i):
            q = wid * _CPWC + 2 * k + i
            return jnp.minimum(q * _CHC, n - _CHC)

        def step(k, _):
            for i in (0, 1):
                start = chunk_start(k, i)

                @pl.when(k > 0)
                def _():
                    pltpu.make_async_copy(
                        rv[i], out_hbm.at[pl.ds(start, _CHC)], ws[i]).wait()

                pltpu.sync_copy(idx_hbm.at[pl.ds(start, _CHC)], iv[i])
                pltpu.async_copy(table_hbm.at[iv[i]], rv[i], gs[i])
            for i in (0, 1):
                start = chunk_start(k, i)
                pltpu.make_async_copy(table_hbm.at[iv[i]], rv[i], gs[i]).wait()
                pltpu.async_copy(rv[i], out_hbm.at[pl.ds(start, _CHC)], ws[i])
            return ()

        lax.fori_loop(0, _CPWC // 2, step, ())
        for i in (0, 1):
            pltpu.make_async_copy(
                rv[i], out_hbm.at[pl.ds(chunk_start(_CPWC // 2 - 1, i), _CHC)],
                ws[i]).wait()

    return collect_kernel(table, slot)


def _grouped_matmul(sx, se, block_type, W16, b):
    def mm_kernel(bt_ref, a1_ref, a2_ref, w_ref, b_ref, o_ref):
        t_id = bt_ref[pl.program_id(0)]
        w = w_ref[t_id]
        a1 = a1_ref[...].astype(jnp.bfloat16)
        a2 = a2_ref[...].astype(jnp.bfloat16)
        dn = (((1,), (1,)), ((), ()))
        acc = lax.dot_general(a1, w[:, :D_HALF], dn,
                              preferred_element_type=jnp.float32)
        acc = acc + lax.dot_general(a2, w[:, D_HALF:], dn,
                                    preferred_element_type=jnp.float32)
        o_ref[...] = jnp.maximum(acc + b_ref[t_id], 0.0)

    grid_spec = pltpu.PrefetchScalarGridSpec(
        num_scalar_prefetch=1,
        grid=(NB,),
        in_specs=[
            pl.BlockSpec((TM, D_HALF), lambda i, bt: (i, 0)),
            pl.BlockSpec((TM, D_HALF), lambda i, bt: (i, 0)),
            pl.BlockSpec((N_TYPES, D_HALF, 2 * D_HALF), lambda i, bt: (0, 0, 0)),
            pl.BlockSpec((N_TYPES, 1, D_HALF), lambda i, bt: (0, 0, 0)),
        ],
        out_specs=pl.BlockSpec((TM, D_HALF), lambda i, bt: (i, 0)),
    )
    return pl.pallas_call(
        mm_kernel,
        grid_spec=grid_spec,
        out_shape=jax.ShapeDtypeStruct((MP, D_HALF), jnp.float32),
    )(block_type, sx, se, W16, b.reshape(N_TYPES, 1, D_HALF))


def kernel(x, edge_attr, node_types, W, b):
    n = x.shape[0]
    t = node_types.astype(jnp.int32)

    nchunks = -(-n // _RANK_S)
    np2 = nchunks * _RANK_S
    t_pad = jnp.pad(t, (0, np2 - n), constant_values=N_TYPES)
    ohf = (t_pad[None, :] == jnp.arange(N_TYPES, dtype=jnp.int32)[:, None]
           ).astype(jnp.float32).reshape(N_TYPES, nchunks, _RANK_S)
    ar = jnp.arange(_RANK_S, dtype=jnp.int32)
    tri = (ar[:, None] < ar[None, :]).astype(jnp.float32)
    local_rank = lax.dot_general(
        ohf, tri, (((2,), (0,)), ((), ())),
        precision=lax.Precision.HIGHEST)
    chunk_cnt = ohf.sum(axis=2)
    chunk_base = jnp.cumsum(chunk_cnt, axis=1) - chunk_cnt
    counts = chunk_cnt.sum(axis=1).astype(jnp.int32)
    padded = ((counts + TM - 1) // TM) * TM
    pstart = (jnp.cumsum(padded) - padded).astype(jnp.float32)
    slot_f = ((local_rank + chunk_base[:, :, None]
               + pstart[:, None, None]) * ohf).sum(axis=0)
    slot = slot_f.reshape(np2).astype(jnp.int32)
    bend = (jnp.cumsum(padded) // TM).astype(jnp.int32)
    block_type = jnp.minimum(
        jnp.searchsorted(bend, jnp.arange(NB, dtype=jnp.int32), side="right"),
        N_TYPES - 1,
    ).astype(jnp.int32)

    sx, se = _dispatch(x, edge_attr, slot)

    out_sorted = _grouped_matmul(sx, se, block_type, W.astype(jnp.bfloat16), b)

    return _collect(out_sorted, slot, n)


# speedup vs baseline: 1.1156x; 1.1156x over previous
import functools

import jax
import jax.numpy as jnp
from jax import lax
from jax.experimental import pallas as pl
from jax.experimental.pallas import tpu as pltpu
from jax.experimental.pallas import tpu_sc as plsc

N_TYPES = 17
D_HALF = 512
TM = 512
NB = 115
MP = NB * TM

_NC = 2
_NS = 16
_NW = _NC * _NS

_CHD = 56
_CPWD = 28
_CHC = 112
_CPWC = 14

_RANK_S = 128


def _dispatch(x, edge_attr, slot):
    n = x.shape[0]
    mesh = plsc.VectorSubcoreMesh(core_axis_name="c", subcore_axis_name="s")

    @functools.partial(
        pl.kernel,
        mesh=mesh,
        out_type=(
            jax.ShapeDtypeStruct((MP, D_HALF), jnp.float32),
            jax.ShapeDtypeStruct((MP, D_HALF), jnp.float32),
        ),
        scratch_types=[
            pltpu.VMEM((_CHD,), jnp.int32),
            pltpu.VMEM((_CHD,), jnp.int32),
            pltpu.VMEM((_CHD, D_HALF), jnp.float32),
            pltpu.VMEM((_CHD, D_HALF), jnp.float32),
            pltpu.VMEM((_CHD, D_HALF), jnp.float32),
            pltpu.VMEM((_CHD, D_HALF), jnp.float32),
            pltpu.SemaphoreType.DMA,
            pltpu.SemaphoreType.DMA,
            pltpu.SemaphoreType.DMA,
            pltpu.SemaphoreType.DMA,
        ],
    )
    def dispatch_kernel(x_hbm, e_hbm, idx_hbm, sx_hbm, se_hbm,
                        iv0, iv1, rx0, rx1, re0, re1, sx0, sx1, se0, se1):
        wid = lax.axis_index("s") * _NC + lax.axis_index("c")
        iv, rx, re = [iv0, iv1], [rx0, rx1], [re0, re1]
        ssx, sse = [sx0, sx1], [se0, se1]

        def step(k, _):
            for i in (0, 1):
                q = wid * _CPWD + 2 * k + i
                start = jnp.minimum(q * _CHD, n - _CHD)

                @pl.when(k > 0)
                def _():
                    pltpu.make_async_copy(rx[i], sx_hbm.at[iv[i]], ssx[i]).wait()
                    pltpu.make_async_copy(re[i], se_hbm.at[iv[i]], sse[i]).wait()

                pltpu.sync_copy(idx_hbm.at[pl.ds(start, _CHD)], iv[i])
                pltpu.sync_copy(x_hbm.at[pl.ds(start, _CHD)], rx[i])
                pltpu.sync_copy(e_hbm.at[pl.ds(start, _CHD)], re[i])
                pltpu.async_copy(rx[i], sx_hbm.at[iv[i]], ssx[i])
                pltpu.async_copy(re[i], se_hbm.at[iv[i]], sse[i])
            return ()

        lax.fori_loop(0, _CPWD // 2, step, ())
        for i in (0, 1):
            pltpu.make_async_copy(rx[i], sx_hbm.at[iv[i]], ssx[i]).wait()
            pltpu.make_async_copy(re[i], se_hbm.at[iv[i]], sse[i]).wait()

    return dispatch_kernel(x, edge_attr, slot)


def _collect(table, slot, n):
    mesh = plsc.VectorSubcoreMesh(core_axis_name="c", subcore_axis_name="s")

    @functools.partial(
        pl.kernel,
        mesh=mesh,
        out_type=jax.ShapeDtypeStruct((n, D_HALF), jnp.float32),
        scratch_types=[
            pltpu.VMEM((_CHC,), jnp.int32),
            pltpu.VMEM((_CHC,), jnp.int32),
            pltpu.VMEM((_CHC, D_HALF), jnp.float32),
            pltpu.VMEM((_CHC, D_HALF), jnp.float32),
            pltpu.SemaphoreType.DMA,
            pltpu.SemaphoreType.DMA,
            pltpu.SemaphoreType.DMA,
            pltpu.SemaphoreType.DMA,
        ],
    )
    def collect_kernel(table_hbm, idx_hbm, out_hbm,
                       iv0, iv1, rv0, rv1, g0, g1, w0, w1):
        wid = lax.axis_index("s") * _NC + lax.axis_index("c")
        iv, rv = [iv0, iv1], [rv0, rv1]
        gs, ws = [g0, g1], [w0, w1]

        def chunk_start(k, i):
            q = wid * _CPWC + 2 * k + i
            return jnp.minimum(q * _CHC, n - _CHC)

        def step(k, _):
            for i in (0, 1):
                start = chunk_start(k, i)

                @pl.when(k > 0)
                def _():
                    pltpu.make_async_copy(
                        rv[i], out_hbm.at[pl.ds(start, _CHC)], ws[i]).wait()

                pltpu.sync_copy(idx_hbm.at[pl.ds(start, _CHC)], iv[i])
                pltpu.async_copy(table_hbm.at[iv[i]], rv[i], gs[i])
            for i in (0, 1):
                start = chunk_start(k, i)
                pltpu.make_async_copy(table_hbm.at[iv[i]], rv[i], gs[i]).wait()
                pltpu.async_copy(rv[i], out_hbm.at[pl.ds(start, _CHC)], ws[i])
            return ()

        lax.fori_loop(0, _CPWC // 2, step, ())
        for i in (0, 1):
            pltpu.make_async_copy(
                rv[i], out_hbm.at[pl.ds(chunk_start(_CPWC // 2 - 1, i), _CHC)],
                ws[i]).wait()

    return collect_kernel(table, slot)


def _grouped_matmul(sx, se, block_type, W16, b):
    def mm_kernel(bt_ref, a1_ref, a2_ref, w_ref, b_ref, o_ref):
        t_id = bt_ref[pl.program_id(0)]
        w = w_ref[t_id]
        a1 = a1_ref[...].astype(jnp.bfloat16)
        a2 = a2_ref[...].astype(jnp.bfloat16)
        dn = (((1,), (1,)), ((), ()))
        acc = lax.dot_general(a1, w[:, :D_HALF], dn,
                              preferred_element_type=jnp.float32)
        acc = acc + lax.dot_general(a2, w[:, D_HALF:], dn,
                                    preferred_element_type=jnp.float32)
        o_ref[...] = jnp.maximum(acc + b_ref[t_id], 0.0)

    grid_spec = pltpu.PrefetchScalarGridSpec(
        num_scalar_prefetch=1,
        grid=(NB,),
        in_specs=[
            pl.BlockSpec((TM, D_HALF), lambda i, bt: (i, 0)),
            pl.BlockSpec((TM, D_HALF), lambda i, bt: (i, 0)),
            pl.BlockSpec((N_TYPES, D_HALF, 2 * D_HALF), lambda i, bt: (0, 0, 0)),
            pl.BlockSpec((N_TYPES, 1, D_HALF), lambda i, bt: (0, 0, 0)),
        ],
        out_specs=pl.BlockSpec((TM, D_HALF), lambda i, bt: (i, 0)),
    )
    return pl.pallas_call(
        mm_kernel,
        grid_spec=grid_spec,
        out_shape=jax.ShapeDtypeStruct((MP, D_HALF), jnp.float32),
    )(block_type, sx, se, W16, b.reshape(N_TYPES, 1, D_HALF))


def kernel(x, edge_attr, node_types, W, b):
    n = x.shape[0]
    t = node_types.astype(jnp.int32)

    nchunks = -(-n // _RANK_S)
    np2 = nchunks * _RANK_S
    t_pad = jnp.pad(t, (0, np2 - n), constant_values=N_TYPES)
    ohf = (t_pad[None, :] == jnp.arange(N_TYPES, dtype=jnp.int32)[:, None]
           ).astype(jnp.float32).reshape(N_TYPES, nchunks, _RANK_S)
    ar = jnp.arange(_RANK_S, dtype=jnp.int32)
    tri = (ar[:, None] < ar[None, :]).astype(jnp.float32)
    local_rank = lax.dot_general(
        ohf, tri, (((2,), (0,)), ((), ())),
        precision=lax.Precision.HIGHEST)
    chunk_cnt = ohf.sum(axis=2)
    chunk_base = jnp.cumsum(chunk_cnt, axis=1) - chunk_cnt
    counts = chunk_cnt.sum(axis=1).astype(jnp.int32)
    padded = ((counts + TM - 1) // TM) * TM
    pstart = (jnp.cumsum(padded) - padded).astype(jnp.float32)
    slot_f = ((local_rank + chunk_base[:, :, None]
               + pstart[:, None, None]) * ohf).sum(axis=0)
    slot = slot_f.reshape(np2).astype(jnp.int32)
    bend = (jnp.cumsum(padded) // TM).astype(jnp.int32)
    block_type = jnp.minimum(
        jnp.searchsorted(bend, jnp.arange(NB, dtype=jnp.int32), side="right"),
        N_TYPES - 1,
    ).astype(jnp.int32)

    sx, se = _dispatch(x, edge_attr, slot)

    out_sorted = _grouped_matmul(sx, se, block_type, W.astype(jnp.bfloat16), b)

    return _collect(out_sorted, slot, n)
